# baseline (speedup 1.0000x reference)
import functools

import jax
import jax.numpy as jnp
from jax.experimental import pallas as pl
from jax.experimental.pallas import tpu as pltpu

B, T, IN, H = 64, 128, 1024, 2048
FH = 4 * H
TB = T * B
BH = B // 2


def _gemm_bias_kernel(x_ref, w_ref, b_ref, o_ref):
    o_ref[...] = (
        jnp.dot(x_ref[...], w_ref[...], preferred_element_type=jnp.float32)
        + b_ref[...]
    ).astype(jnp.bfloat16)


U = 2
NCHUNK = 1
TCH = T // NCHUNK


def _lstm_step_kernel(yx_ref, wat_ref, a0_ref, c0_ref,
                      a_out, c_out, yi_out, yf_out, yg_out, yo_out,
                      a_last, c_last, a_scr, c_scr):
    t = pl.program_id(0)

    @pl.when(t == 0)
    def _init():
        a_scr[...] = a0_ref[...]
        c_scr[...] = c0_ref[...]

    a_prev = a_scr[...]
    c_prev = c_scr[...]
    for s in range(U):
        y = yx_ref[s * B:(s + 1) * B, :] + jnp.dot(
            a_prev.astype(jnp.bfloat16), wat_ref[...],
            preferred_element_type=jnp.float32)
        yi = y[:, 0 * H:1 * H]
        yf = y[:, 1 * H:2 * H]
        yg = y[:, 2 * H:3 * H]
        yo = y[:, 3 * H:4 * H]
        c_t = jax.nn.sigmoid(yf) * c_prev + jax.nn.sigmoid(yi) * jnp.tanh(yg)
        a_t = jax.nn.sigmoid(yo) * jnp.tanh(c_t)
        hs = slice(s * H, (s + 1) * H)
        yi_out[:, hs] = yi
        yf_out[:, hs] = yf
        yg_out[:, hs] = yg
        yo_out[:, hs] = yo
        c_out[:, hs] = c_t
        a_out[:, hs] = a_t
        a_prev, c_prev = a_t, c_t
    a_scr[...] = a_prev
    c_scr[...] = c_prev

    @pl.when(t == TCH // U - 1)
    def _fin():
        a_last[...] = a_prev
        c_last[...] = c_prev


RT = 8
NTC = 2


def _relayout_kernel(*refs):
    n = len(refs) // 2
    for x_ref, o_ref in zip(refs[:n], refs[n:]):
        o_ref[...] = x_ref[...].reshape(B, RT, H)


def _to_bth_tc(flats):
    outs = []
    for k in range(0, len(flats), 2):
        group = flats[k:k + 2]
        n = len(group)
        res = pl.pallas_call(
            _relayout_kernel,
            grid=(T // RT,),
            in_specs=[pl.BlockSpec((B, RT * H), lambda i: (0, i))] * n,
            out_specs=[pl.BlockSpec((B, RT, H), lambda i: (0, i, 0))] * n,
            out_shape=[jax.ShapeDtypeStruct((B, T, H), jnp.float32)] * n,
            compiler_params=pltpu.CompilerParams(
                dimension_semantics=("arbitrary",)),
        )(*group)
        outs.extend(res if n > 1 else [res])
    return outs


@jax.jit
def kernel(x, Wx, Wa, b, a0, c0):
    x_tm = jnp.swapaxes(x, 0, 1).reshape(TB, IN).astype(jnp.bfloat16)
    wxt = Wx.T.astype(jnp.bfloat16)
    b2 = b.reshape(1, FH)

    BM, BN = 2048, 1024
    yx = pl.pallas_call(
        _gemm_bias_kernel,
        grid=(TB // BM, FH // BN),
        in_specs=[
            pl.BlockSpec((BM, IN), lambda i, j: (i, 0)),
            pl.BlockSpec((IN, BN), lambda i, j: (0, j)),
            pl.BlockSpec((1, BN), lambda i, j: (0, j)),
        ],
        out_specs=pl.BlockSpec((BM, BN), lambda i, j: (i, j)),
        out_shape=jax.ShapeDtypeStruct((TB, FH), jnp.bfloat16),
        compiler_params=pltpu.CompilerParams(
            dimension_semantics=("parallel", "arbitrary")),
    )(x_tm, wxt, b2)

    wat = Wa.T.astype(jnp.bfloat16)

    out_sd = jax.ShapeDtypeStruct((B, TCH * H), jnp.float32)
    carry_sd = jax.ShapeDtypeStruct((B, H), jnp.float32)
    out_spec = pl.BlockSpec((B, U * H), lambda t: (0, t))
    carry_spec = pl.BlockSpec((B, H), lambda t: (0, 0))

    a_c, c_c = a0, c0
    parts = []
    for ci in range(NCHUNK):
        row0 = ci * (TCH // U)
        *outs, a_c, c_c = pl.pallas_call(
            _lstm_step_kernel,
            grid=(TCH // U,),
            in_specs=[
                pl.BlockSpec((U * B, FH), lambda t, row0=row0: (row0 + t, 0)),
                pl.BlockSpec((H, FH), lambda t: (0, 0)),
                carry_spec,
                carry_spec,
            ],
            out_specs=[out_spec] * 6 + [carry_spec] * 2,
            out_shape=[out_sd] * 6 + [carry_sd] * 2,
            scratch_shapes=[
                pltpu.VMEM((B, H), jnp.float32),
                pltpu.VMEM((B, H), jnp.float32),
            ],
            compiler_params=pltpu.CompilerParams(
                dimension_semantics=("arbitrary",)),
        )(yx, wat, a_c, c_c)
        parts.append(outs)

    flats = [parts[0][i] for i in range(6)]
    tc_outs = _to_bth_tc(flats[:NTC])
    sc_outs = [o.reshape(B, T, H) for o in flats[NTC:]]
    return tuple(tc_outs) + tuple(sc_outs)


# speedup vs baseline: 1.0463x; 1.0463x over previous
import functools

import jax
import jax.numpy as jnp
from jax.experimental import pallas as pl
from jax.experimental.pallas import tpu as pltpu

B, T, IN, H = 64, 128, 1024, 2048
FH = 4 * H
TB = T * B
BH = B // 2


def _gemm_bias_kernel(x_ref, w_ref, b_ref, o_ref):
    o_ref[...] = (
        jnp.dot(x_ref[...], w_ref[...], preferred_element_type=jnp.float32)
        + b_ref[...]
    ).astype(jnp.bfloat16)


U = 2
NCHUNK = 1
TCH = T // NCHUNK


def _lstm_step_kernel(yx_ref, wat_ref, a0_ref, c0_ref,
                      a_out, c_out, yi_out, yf_out, yg_out, yo_out,
                      a_last, c_last, a_scr, c_scr):
    t = pl.program_id(0)

    @pl.when(t == 0)
    def _init():
        a_scr[...] = a0_ref[...]
        c_scr[...] = c0_ref[...]

    a_prev = a_scr[...]
    c_prev = c_scr[...]
    for s in range(U):
        y = yx_ref[s * B:(s + 1) * B, :] + jnp.dot(
            a_prev.astype(jnp.bfloat16), wat_ref[...],
            preferred_element_type=jnp.float32)
        yi = y[:, 0 * H:1 * H]
        yf = y[:, 1 * H:2 * H]
        yg = y[:, 2 * H:3 * H]
        yo = y[:, 3 * H:4 * H]
        c_t = jax.nn.sigmoid(yf) * c_prev + jax.nn.sigmoid(yi) * jnp.tanh(yg)
        a_t = jax.nn.sigmoid(yo) * jnp.tanh(c_t)
        hs = slice(s * H, (s + 1) * H)
        yi_out[:, hs] = yi.astype(jnp.bfloat16)
        yf_out[:, hs] = yf.astype(jnp.bfloat16)
        yg_out[:, hs] = yg
        yo_out[:, hs] = yo
        c_out[:, hs] = c_t.astype(jnp.bfloat16)
        a_out[:, hs] = a_t.astype(jnp.bfloat16)
        a_prev, c_prev = a_t, c_t
    a_scr[...] = a_prev
    c_scr[...] = c_prev

    @pl.when(t == TCH // U - 1)
    def _fin():
        a_last[...] = a_prev
        c_last[...] = c_prev


RT = 8
NTC = 4


def _relayout_kernel(*refs):
    n = len(refs) // 2
    for x_ref, o_ref in zip(refs[:n], refs[n:]):
        o_ref[...] = x_ref[...].reshape(B, RT, H).astype(jnp.float32)


def _to_bth_tc(flats):
    outs = []
    for k in range(0, len(flats), 2):
        group = flats[k:k + 2]
        n = len(group)
        res = pl.pallas_call(
            _relayout_kernel,
            grid=(T // RT,),
            in_specs=[pl.BlockSpec((B, RT * H), lambda i: (0, i))] * n,
            out_specs=[pl.BlockSpec((B, RT, H), lambda i: (0, i, 0))] * n,
            out_shape=[jax.ShapeDtypeStruct((B, T, H), jnp.float32)] * n,
            compiler_params=pltpu.CompilerParams(
                dimension_semantics=("arbitrary",)),
        )(*group)
        outs.extend(res if n > 1 else [res])
    return outs


@jax.jit
def kernel(x, Wx, Wa, b, a0, c0):
    x_tm = jnp.swapaxes(x, 0, 1).reshape(TB, IN).astype(jnp.bfloat16)
    wxt = Wx.T.astype(jnp.bfloat16)
    b2 = b.reshape(1, FH)

    BM, BN = 2048, 1024
    yx = pl.pallas_call(
        _gemm_bias_kernel,
        grid=(TB // BM, FH // BN),
        in_specs=[
            pl.BlockSpec((BM, IN), lambda i, j: (i, 0)),
            pl.BlockSpec((IN, BN), lambda i, j: (0, j)),
            pl.BlockSpec((1, BN), lambda i, j: (0, j)),
        ],
        out_specs=pl.BlockSpec((BM, BN), lambda i, j: (i, j)),
        out_shape=jax.ShapeDtypeStruct((TB, FH), jnp.bfloat16),
        compiler_params=pltpu.CompilerParams(
            dimension_semantics=("parallel", "arbitrary")),
    )(x_tm, wxt, b2)

    wat = Wa.T.astype(jnp.bfloat16)

    out_bf = jax.ShapeDtypeStruct((B, TCH * H), jnp.bfloat16)
    out_f32 = jax.ShapeDtypeStruct((B, TCH * H), jnp.float32)
    carry_sd = jax.ShapeDtypeStruct((B, H), jnp.float32)
    out_spec = pl.BlockSpec((B, U * H), lambda t: (0, t))
    carry_spec = pl.BlockSpec((B, H), lambda t: (0, 0))

    a_c, c_c = a0, c0
    parts = []
    for ci in range(NCHUNK):
        row0 = ci * (TCH // U)
        *outs, a_c, c_c = pl.pallas_call(
            _lstm_step_kernel,
            grid=(TCH // U,),
            in_specs=[
                pl.BlockSpec((U * B, FH), lambda t, row0=row0: (row0 + t, 0)),
                pl.BlockSpec((H, FH), lambda t: (0, 0)),
                carry_spec,
                carry_spec,
            ],
            out_specs=[out_spec] * 6 + [carry_spec] * 2,
            out_shape=[out_bf] * 4 + [out_f32] * 2 + [carry_sd] * 2,
            scratch_shapes=[
                pltpu.VMEM((B, H), jnp.float32),
                pltpu.VMEM((B, H), jnp.float32),
            ],
            compiler_params=pltpu.CompilerParams(
                dimension_semantics=("arbitrary",)),
        )(yx, wat, a_c, c_c)
        parts.append(outs)

    flats = [parts[0][i] for i in range(6)]
    tc_outs = _to_bth_tc(flats[:NTC])
    sc_outs = [o.reshape(B, T, H) for o in flats[NTC:]]
    return tuple(tc_outs) + tuple(sc_outs)


# speedup vs baseline: 1.0727x; 1.0252x over previous
import functools

import jax
import jax.numpy as jnp
from jax.experimental import pallas as pl
from jax.experimental.pallas import tpu as pltpu

B, T, IN, H = 64, 128, 1024, 2048
FH = 4 * H
TB = T * B
BH = B // 2


def _gemm_bias_kernel(x_ref, w_ref, b_ref, o_ref):
    o_ref[...] = (
        jnp.dot(x_ref[...], w_ref[...], preferred_element_type=jnp.float32)
        + b_ref[...]
    ).astype(jnp.bfloat16)


U = 2
NCHUNK = 1
TCH = T // NCHUNK


def _lstm_step_kernel(yx_ref, wat_ref, a0_ref, c0_ref,
                      a_out, c_out, yi_out, yf_out, yg_out, yo_out,
                      a_last, c_last, a_scr, c_scr):
    t = pl.program_id(0)

    @pl.when(t == 0)
    def _init():
        a_scr[...] = a0_ref[...]
        c_scr[...] = c0_ref[...]

    a_prev = a_scr[...]
    c_prev = c_scr[...]
    for s in range(U):
        y = yx_ref[s * B:(s + 1) * B, :] + jnp.dot(
            a_prev.astype(jnp.bfloat16), wat_ref[...],
            preferred_element_type=jnp.float32)
        yi = y[:, 0 * H:1 * H]
        yf = y[:, 1 * H:2 * H]
        yg = y[:, 2 * H:3 * H]
        yo = y[:, 3 * H:4 * H]
        c_t = jax.nn.sigmoid(yf) * c_prev + jax.nn.sigmoid(yi) * jnp.tanh(yg)
        a_t = jax.nn.sigmoid(yo) * jnp.tanh(c_t)
        hs = slice(s * H, (s + 1) * H)
        yi_out[:, hs] = yi.astype(jnp.bfloat16)
        yf_out[:, hs] = yf.astype(jnp.bfloat16)
        yg_out[:, hs] = yg.astype(jnp.bfloat16)
        yo_out[:, hs] = yo.astype(jnp.bfloat16)
        c_out[:, hs] = c_t.astype(jnp.bfloat16)
        a_out[:, hs] = a_t.astype(jnp.bfloat16)
        a_prev, c_prev = a_t, c_t
    a_scr[...] = a_prev
    c_scr[...] = c_prev

    @pl.when(t == TCH // U - 1)
    def _fin():
        a_last[...] = a_prev
        c_last[...] = c_prev


RT = 8
NTC = 6


def _relayout_kernel(*refs):
    n = len(refs) // 2
    for x_ref, o_ref in zip(refs[:n], refs[n:]):
        o_ref[...] = x_ref[...].reshape(B, RT, H).astype(jnp.float32)


def _to_bth_tc(flats):
    outs = []
    for k in range(0, len(flats), 2):
        group = flats[k:k + 2]
        n = len(group)
        res = pl.pallas_call(
            _relayout_kernel,
            grid=(T // RT,),
            in_specs=[pl.BlockSpec((B, RT * H), lambda i: (0, i))] * n,
            out_specs=[pl.BlockSpec((B, RT, H), lambda i: (0, i, 0))] * n,
            out_shape=[jax.ShapeDtypeStruct((B, T, H), jnp.float32)] * n,
            compiler_params=pltpu.CompilerParams(
                dimension_semantics=("arbitrary",)),
        )(*group)
        outs.extend(res if n > 1 else [res])
    return outs


@jax.jit
def kernel(x, Wx, Wa, b, a0, c0):
    x_tm = jnp.swapaxes(x, 0, 1).reshape(TB, IN).astype(jnp.bfloat16)
    wxt = Wx.T.astype(jnp.bfloat16)
    b2 = b.reshape(1, FH)

    BM, BN = 2048, 1024
    yx = pl.pallas_call(
        _gemm_bias_kernel,
        grid=(TB // BM, FH // BN),
        in_specs=[
            pl.BlockSpec((BM, IN), lambda i, j: (i, 0)),
            pl.BlockSpec((IN, BN), lambda i, j: (0, j)),
            pl.BlockSpec((1, BN), lambda i, j: (0, j)),
        ],
        out_specs=pl.BlockSpec((BM, BN), lambda i, j: (i, j)),
        out_shape=jax.ShapeDtypeStruct((TB, FH), jnp.bfloat16),
        compiler_params=pltpu.CompilerParams(
            dimension_semantics=("parallel", "arbitrary")),
    )(x_tm, wxt, b2)

    wat = Wa.T.astype(jnp.bfloat16)

    out_bf = jax.ShapeDtypeStruct((B, TCH * H), jnp.bfloat16)
    out_f32 = jax.ShapeDtypeStruct((B, TCH * H), jnp.float32)
    carry_sd = jax.ShapeDtypeStruct((B, H), jnp.float32)
    out_spec = pl.BlockSpec((B, U * H), lambda t: (0, t))
    carry_spec = pl.BlockSpec((B, H), lambda t: (0, 0))

    a_c, c_c = a0, c0
    parts = []
    for ci in range(NCHUNK):
        row0 = ci * (TCH // U)
        *outs, a_c, c_c = pl.pallas_call(
            _lstm_step_kernel,
            grid=(TCH // U,),
            in_specs=[
                pl.BlockSpec((U * B, FH), lambda t, row0=row0: (row0 + t, 0)),
                pl.BlockSpec((H, FH), lambda t: (0, 0)),
                carry_spec,
                carry_spec,
            ],
            out_specs=[out_spec] * 6 + [carry_spec] * 2,
            out_shape=[out_bf] * 6 + [carry_sd] * 2,
            scratch_shapes=[
                pltpu.VMEM((B, H), jnp.float32),
                pltpu.VMEM((B, H), jnp.float32),
            ],
            compiler_params=pltpu.CompilerParams(
                dimension_semantics=("arbitrary",)),
        )(yx, wat, a_c, c_c)
        parts.append(outs)

    flats = [parts[0][i] for i in range(6)]
    tc_outs = _to_bth_tc(flats[:NTC])
    sc_outs = [o.reshape(B, T, H) for o in flats[NTC:]]
    return tuple(tc_outs) + tuple(sc_outs)


# speedup vs baseline: 1.0786x; 1.0055x over previous
import functools

import jax
import jax.numpy as jnp
from jax.experimental import pallas as pl
from jax.experimental.pallas import tpu as pltpu

B, T, IN, H = 64, 128, 1024, 2048
FH = 4 * H
TB = T * B
BH = B // 2


def _gemm_bias_kernel(x_ref, w_ref, b_ref, o_ref):
    o_ref[...] = (
        jnp.dot(x_ref[...], w_ref[...], preferred_element_type=jnp.float32)
        + b_ref[...]
    ).astype(jnp.bfloat16)


U = 2
NCHUNK = 1
TCH = T // NCHUNK


def _lstm_step_kernel(yx_ref, wat_ref, a0_ref, c0_ref,
                      a_out, c_out, yi_out, yf_out, yg_out, yo_out,
                      a_last, c_last, a_scr, c_scr):
    t = pl.program_id(0)

    @pl.when(t == 0)
    def _init():
        a_scr[...] = a0_ref[...]
        c_scr[...] = c0_ref[...]

    a_prev = a_scr[...]
    c_prev = c_scr[...]
    for s in range(U):
        y = yx_ref[s * B:(s + 1) * B, :] + jnp.dot(
            a_prev.astype(jnp.bfloat16), wat_ref[...],
            preferred_element_type=jnp.float32)
        yi = y[:, 0 * H:1 * H]
        yf = y[:, 1 * H:2 * H]
        yg = y[:, 2 * H:3 * H]
        yo = y[:, 3 * H:4 * H]
        c_t = jax.nn.sigmoid(yf) * c_prev + jax.nn.sigmoid(yi) * jnp.tanh(yg)
        a_t = jax.nn.sigmoid(yo) * jnp.tanh(c_t)
        hs = slice(s * H, (s + 1) * H)
        yi_out[:, hs] = yi.astype(jnp.bfloat16)
        yf_out[:, hs] = yf.astype(jnp.bfloat16)
        yg_out[:, hs] = yg.astype(jnp.bfloat16)
        yo_out[:, hs] = yo.astype(jnp.bfloat16)
        c_out[:, hs] = c_t.astype(jnp.bfloat16)
        a_out[:, hs] = a_t.astype(jnp.bfloat16)
        a_prev, c_prev = a_t, c_t
    a_scr[...] = a_prev
    c_scr[...] = c_prev

    @pl.when(t == TCH // U - 1)
    def _fin():
        a_last[...] = a_prev
        c_last[...] = c_prev


RT = 8
NTC = 5


def _relayout_kernel(*refs):
    n = len(refs) // 2
    for x_ref, o_ref in zip(refs[:n], refs[n:]):
        o_ref[...] = x_ref[...].reshape(B, RT, H).astype(jnp.float32)


def _to_bth_tc(flats):
    outs = []
    for k in range(0, len(flats), 2):
        group = flats[k:k + 2]
        n = len(group)
        res = pl.pallas_call(
            _relayout_kernel,
            grid=(T // RT,),
            in_specs=[pl.BlockSpec((B, RT * H), lambda i: (0, i))] * n,
            out_specs=[pl.BlockSpec((B, RT, H), lambda i: (0, i, 0))] * n,
            out_shape=[jax.ShapeDtypeStruct((B, T, H), jnp.float32)] * n,
            compiler_params=pltpu.CompilerParams(
                dimension_semantics=("arbitrary",)),
        )(*group)
        outs.extend(res if n > 1 else [res])
    return outs


@jax.jit
def kernel(x, Wx, Wa, b, a0, c0):
    x_tm = jnp.swapaxes(x, 0, 1).reshape(TB, IN).astype(jnp.bfloat16)
    wxt = Wx.T.astype(jnp.bfloat16)
    b2 = b.reshape(1, FH)

    BM, BN = 2048, 1024
    yx = pl.pallas_call(
        _gemm_bias_kernel,
        grid=(TB // BM, FH // BN),
        in_specs=[
            pl.BlockSpec((BM, IN), lambda i, j: (i, 0)),
            pl.BlockSpec((IN, BN), lambda i, j: (0, j)),
            pl.BlockSpec((1, BN), lambda i, j: (0, j)),
        ],
        out_specs=pl.BlockSpec((BM, BN), lambda i, j: (i, j)),
        out_shape=jax.ShapeDtypeStruct((TB, FH), jnp.bfloat16),
        compiler_params=pltpu.CompilerParams(
            dimension_semantics=("parallel", "arbitrary")),
    )(x_tm, wxt, b2)

    wat = Wa.T.astype(jnp.bfloat16)

    out_bf = jax.ShapeDtypeStruct((B, TCH * H), jnp.bfloat16)
    out_f32 = jax.ShapeDtypeStruct((B, TCH * H), jnp.float32)
    carry_sd = jax.ShapeDtypeStruct((B, H), jnp.float32)
    out_spec = pl.BlockSpec((B, U * H), lambda t: (0, t))
    carry_spec = pl.BlockSpec((B, H), lambda t: (0, 0))

    a_c, c_c = a0, c0
    parts = []
    for ci in range(NCHUNK):
        row0 = ci * (TCH // U)
        *outs, a_c, c_c = pl.pallas_call(
            _lstm_step_kernel,
            grid=(TCH // U,),
            in_specs=[
                pl.BlockSpec((U * B, FH), lambda t, row0=row0: (row0 + t, 0)),
                pl.BlockSpec((H, FH), lambda t: (0, 0)),
                carry_spec,
                carry_spec,
            ],
            out_specs=[out_spec] * 6 + [carry_spec] * 2,
            out_shape=[out_bf] * 6 + [carry_sd] * 2,
            scratch_shapes=[
                pltpu.VMEM((B, H), jnp.float32),
                pltpu.VMEM((B, H), jnp.float32),
            ],
            compiler_params=pltpu.CompilerParams(
                dimension_semantics=("arbitrary",)),
        )(yx, wat, a_c, c_c)
        parts.append(outs)

    flats = [parts[0][i] for i in range(6)]
    tc_outs = _to_bth_tc(flats[:NTC])
    sc_outs = [o.reshape(B, T, H) for o in flats[NTC:]]
    return tuple(tc_outs) + tuple(sc_outs)
